# baseline (speedup 1.0000x reference)
import functools

import jax
import jax.numpy as jnp
from jax import lax
from jax.experimental import pallas as pl
from jax.experimental.pallas import tpu as pltpu
from jax.experimental.pallas import tpu_sc as plsc

_VOCAB = 1000000
_D = 16
_B = 4096
_L = 200

_NC = 2
_NS = 16
_NW = _NC * _NS

_CHUNK = 100
_HALVES = _L // _CHUNK
_ROWS_PER_W = _B // _NW
_IDX_ROWS = _ROWS_PER_W * _HALVES


def _sc_kernel(idx_hbm, table_hbm, out_hbm, idx_v, buf_v, out_v, sem):
    wid = lax.axis_index("s") * _NC + lax.axis_index("c")

    pltpu.sync_copy(idx_hbm.at[pl.ds(wid * _IDX_ROWS, _IDX_ROWS), :], idx_v)

    def row_body(r, _):
        def half_body(h, acc):
            pltpu.async_copy(
                table_hbm.at[idx_v.at[r * _HALVES + h]], buf_v, sem
            ).wait()

            def add_body(l, a):
                return a + buf_v[l, :]

            return lax.fori_loop(0, _CHUNK, add_body, acc)

        acc = lax.fori_loop(
            0, _HALVES, half_body, jnp.zeros((_D,), jnp.float32)
        )
        out_v[r, :] = acc * (1.0 / _L)
        return 0

    lax.fori_loop(0, _ROWS_PER_W, row_body, 0)

    pltpu.sync_copy(out_v, out_hbm.at[pl.ds(wid * _ROWS_PER_W, _ROWS_PER_W), :])


@jax.jit
def kernel(indices, table):
    idx2d = indices.reshape(_B * _HALVES, _CHUNK)
    run = functools.partial(
        pl.kernel,
        out_type=jax.ShapeDtypeStruct((_B, _D), jnp.float32),
        mesh=plsc.VectorSubcoreMesh(core_axis_name="c", subcore_axis_name="s"),
        compiler_params=pltpu.CompilerParams(use_tc_tiling_on_sc=False),
        scratch_types=[
            pltpu.VMEM((_IDX_ROWS, _CHUNK), jnp.int32),
            pltpu.VMEM((_CHUNK, _D), jnp.float32),
            pltpu.VMEM((_ROWS_PER_W, _D), jnp.float32),
            pltpu.SemaphoreType.DMA,
        ],
    )(_sc_kernel)
    return run(idx2d, table)


# speedup vs baseline: 1.4088x; 1.4088x over previous
import functools

import jax
import jax.numpy as jnp
from jax import lax
from jax.experimental import pallas as pl
from jax.experimental.pallas import tpu as pltpu
from jax.experimental.pallas import tpu_sc as plsc

_VOCAB = 1000000
_D = 16
_B = 4096
_L = 200

_NC = 2
_NS = 16
_NW = _NC * _NS

_CHUNK = 100
_HALVES = _L // _CHUNK
_ROWS_PER_W = _B // _NW
_IDX_ROWS = _ROWS_PER_W * _HALVES
_NBUF = 8


def _tree_sum(buf):
    vals = [buf[l, :] for l in range(_CHUNK)]
    while len(vals) > 1:
        nxt = [vals[i] + vals[i + 1] for i in range(0, len(vals) - 1, 2)]
        if len(vals) % 2:
            nxt.append(vals[-1])
        vals = nxt
    return vals[0]


def _sc_kernel(idx_hbm, table_hbm, out_hbm, idx_v, out_v, *rest):
    bufs = rest[:_NBUF]
    sems = rest[_NBUF]
    wid = lax.axis_index("s") * _NC + lax.axis_index("c")

    pltpu.sync_copy(idx_hbm.at[pl.ds(wid * _IDX_ROWS, _IDX_ROWS), :], idx_v)

    for b in range(_NBUF):
        pltpu.async_copy(table_hbm.at[idx_v.at[b]], bufs[b], sems.at[b])

    @pl.loop(0, _IDX_ROWS, step=_NBUF)
    def _(j):
        for p in range(_NBUF // 2):
            sums = []
            for b in (2 * p, 2 * p + 1):
                h = j + b
                pltpu.make_async_copy(
                    table_hbm.at[idx_v.at[0]], bufs[b], sems.at[b]
                ).wait()
                sums.append(_tree_sum(bufs[b]))

                @pl.when(h + _NBUF < _IDX_ROWS)
                def _():
                    pltpu.async_copy(
                        table_hbm.at[idx_v.at[h + _NBUF]], bufs[b], sems.at[b]
                    )

            r = j // _HALVES + p
            out_v[r, :] = (sums[0] + sums[1]) * (1.0 / _L)

    pltpu.sync_copy(out_v, out_hbm.at[pl.ds(wid * _ROWS_PER_W, _ROWS_PER_W), :])


@jax.jit
def kernel(indices, table):
    idx2d = indices.reshape(_B * _HALVES, _CHUNK)
    run = functools.partial(
        pl.kernel,
        out_type=jax.ShapeDtypeStruct((_B, _D), jnp.float32),
        mesh=plsc.VectorSubcoreMesh(core_axis_name="c", subcore_axis_name="s"),
        compiler_params=pltpu.CompilerParams(use_tc_tiling_on_sc=False),
        scratch_types=[
            pltpu.VMEM((_IDX_ROWS, _CHUNK), jnp.int32),
            pltpu.VMEM((_ROWS_PER_W, _D), jnp.float32),
        ]
        + [pltpu.VMEM((_CHUNK, _D), jnp.float32) for _ in range(_NBUF)]
        + [pltpu.SemaphoreType.DMA((_NBUF,))],
    )(_sc_kernel)
    return run(idx2d, table)
